# Initial kernel scaffold; baseline (speedup 1.0000x reference)
import jax
import jax.numpy as jnp
from jax.experimental import pallas as pl


def kernel(x, l0_w, l0_b, l0_gamma, l0_beta, l1_w, l1_b, l1_gamma, l1_beta, cls_w, cls_b):
    raise NotImplementedError("write your pallas kernel here")


# speedup vs baseline: 10.7520x; 10.7520x over previous
import functools

import numpy as np

import jax
import jax.numpy as jnp
from jax import lax
from jax.experimental import pallas as pl
from jax.experimental.pallas import tpu as pltpu


def _stage_geom(d, h, w):
    dp, hp, wp = d + 2, h + 2, w + 2
    hwp = hp * wp
    r = (d - 1) * hwp + (h - 1) * wp + (w - 1) + 1
    base = (np.arange(d)[:, None, None] * hwp
            + np.arange(h)[None, :, None] * wp
            + np.arange(w)[None, None, :])
    mask = np.zeros((1, r), np.float32)
    mask[0, base.reshape(-1)] = 1.0 / (d * h * w)
    return dp, hp, wp, r, base, mask


def _pool_mat(base, r, dq, hq, wq, dst_of):
    n_dst = int(np.max(dst_of)) + 1
    p = np.zeros((n_dst, r), np.float32)
    rows = dst_of.reshape(-1)
    for od in range(2):
        for oh in range(2):
            for ow in range(2):
                src = base[od:2 * dq:2, oh:2 * hq:2, ow:2 * wq:2].reshape(-1)
                p[rows, src] = 0.125
    return p


def _net_kernel(xf_ref, w0_ref, p0_ref, m0_ref, g0_ref, b0_ref,
                w1_ref, p1_ref, m1_ref, g1_ref, b1_ref,
                o_ref, acc0_ref, x1_ref, acc1_ref, *, d0, r0, d1, r1):
    for t, dl in enumerate(d0):
        part = jnp.dot(xf_ref[pl.ds(dl, r0), :], w0_ref[t],
                       preferred_element_type=jnp.float32)
        if t == 0:
            acc0_ref[...] = part
        else:
            acc0_ref[...] += part
    acc = acc0_ref[...]
    mean = jnp.dot(m0_ref[...], acc, preferred_element_type=jnp.float32)
    ex2 = jnp.dot(m0_ref[...], acc * acc, preferred_element_type=jnp.float32)
    var = jnp.maximum(ex2 - mean * mean, 0.0)
    scale = g0_ref[...] * lax.rsqrt(var + 1e-5)
    shift = b0_ref[...] - mean * scale
    y = jnp.maximum(acc * scale + shift, 0.0).astype(jnp.bfloat16)
    x1_ref[...] = jnp.dot(p0_ref[...], y,
                          preferred_element_type=jnp.float32).astype(jnp.bfloat16)

    for t, dl in enumerate(d1):
        part = jnp.dot(x1_ref[pl.ds(dl, r1), :], w1_ref[t],
                       preferred_element_type=jnp.float32)
        if t == 0:
            acc1_ref[...] = part
        else:
            acc1_ref[...] += part
    acc1 = acc1_ref[...]
    mean1 = jnp.dot(m1_ref[...], acc1, preferred_element_type=jnp.float32)
    ex21 = jnp.dot(m1_ref[...], acc1 * acc1, preferred_element_type=jnp.float32)
    var1 = jnp.maximum(ex21 - mean1 * mean1, 0.0)
    scale1 = g1_ref[...] * lax.rsqrt(var1 + 1e-5)
    shift1 = b1_ref[...] - mean1 * scale1
    y1 = jnp.maximum(acc1 * scale1 + shift1, 0.0).astype(jnp.bfloat16)
    o_ref[...] = jnp.dot(p1_ref[...], y1, preferred_element_type=jnp.float32)


def _linear_kernel(a_ref, w_ref, b_ref, o_ref):
    o_ref[...] = (jnp.dot(a_ref[...], w_ref[...],
                          preferred_element_type=jnp.float32) + b_ref[...])


def kernel(x, l0_w, l0_b, l0_gamma, l0_beta,
           l1_w, l1_b, l1_gamma, l1_beta, cls_w, cls_b):
    n, c_in, d, h, w = (int(s) for s in x.shape)
    cw = int(l0_w.shape[0])
    grp = 16
    ng = n // grp
    lanes = grp * cw

    dp0, hp0, wp0, r0, base0, mask0 = _stage_geom(d, h, w)
    s0 = dp0 * hp0 * wp0
    d1, h1, w1 = d // 2, h // 2, w // 2
    dp1, hp1, wp1, r1, base1, mask1 = _stage_geom(d1, h1, w1)
    s1 = dp1 * hp1 * wp1
    d2, h2, w2 = d1 // 2, h1 // 2, w1 // 2
    s2 = d2 * h2 * w2

    deltas0 = tuple(kd * (hp0 * wp0) + kh * wp0
                    for kd in range(3) for kh in range(3))
    deltas1 = tuple(kd * (hp1 * wp1) + kh * wp1 + kw
                    for kd in range(3) for kh in range(3) for kw in range(3))

    dst0 = ((np.arange(d1) + 1)[:, None, None] * (hp1 * wp1)
            + (np.arange(h1) + 1)[None, :, None] * wp1
            + (np.arange(w1) + 1)[None, None, :])
    pool0 = np.zeros((s1, r0), np.float32)
    pool0[:dst0.max() + 1] = _pool_mat(base0, r0, d1, h1, w1, dst0)
    dst1 = (np.arange(d2)[:, None, None] * (h2 * w2)
            + np.arange(h2)[None, :, None] * w2
            + np.arange(w2)[None, None, :])
    pool1 = _pool_mat(base1, r1, d2, h2, w2, dst1)

    xp = jnp.transpose(x, (0, 2, 3, 4, 1))
    xp = jnp.pad(xp, ((0, 0), (1, 1), (1, 1), (1, 1), (0, 0)))
    xf = xp.reshape(ng, grp, s0, c_in).transpose(0, 2, 1, 3)
    xf = xf.reshape(ng, s0, grp * c_in)
    xf = jnp.pad(xf, ((0, 0), (0, 2), (0, 0)))
    xf3 = jnp.concatenate([xf[:, 0:s0], xf[:, 1:s0 + 1], xf[:, 2:s0 + 2]],
                          axis=2).astype(jnp.bfloat16)

    eye = jnp.eye(grp, dtype=jnp.float32)
    w0t = jnp.transpose(l0_w, (2, 3, 4, 1, 0)).reshape(9, 3, c_in, cw)
    w0bd = jnp.einsum('gh,tkio->tkgiho', eye, w0t)
    w0bd = w0bd.reshape(9, 3 * grp * c_in, lanes).astype(jnp.bfloat16)
    w1t = jnp.transpose(l1_w, (2, 3, 4, 1, 0)).reshape(27, cw, cw)
    w1bd = jnp.einsum('gh,tio->tgiho', eye, w1t)
    w1bd = w1bd.reshape(27, grp * cw, lanes).astype(jnp.bfloat16)

    g0 = jnp.tile(l0_gamma, grp).reshape(1, lanes)
    bt0 = jnp.tile(l0_beta, grp).reshape(1, lanes)
    g1 = jnp.tile(l1_gamma, grp).reshape(1, lanes)
    bt1 = jnp.tile(l1_beta, grp).reshape(1, lanes)

    k_in = 3 * grp * c_in
    _body = functools.partial(_net_kernel, d0=deltas0, r0=r0,
                              d1=deltas1, r1=r1)
    y = pl.pallas_call(
        _body,
        out_shape=jax.ShapeDtypeStruct((ng, s2, lanes), jnp.float32),
        grid=(ng,),
        in_specs=[
            pl.BlockSpec((None, s0, k_in), lambda i: (i, 0, 0)),
            pl.BlockSpec((9, k_in, lanes), lambda i: (0, 0, 0)),
            pl.BlockSpec((s1, r0), lambda i: (0, 0)),
            pl.BlockSpec((1, r0), lambda i: (0, 0)),
            pl.BlockSpec((1, lanes), lambda i: (0, 0)),
            pl.BlockSpec((1, lanes), lambda i: (0, 0)),
            pl.BlockSpec((27, grp * cw, lanes), lambda i: (0, 0, 0)),
            pl.BlockSpec((s2, r1), lambda i: (0, 0)),
            pl.BlockSpec((1, r1), lambda i: (0, 0)),
            pl.BlockSpec((1, lanes), lambda i: (0, 0)),
            pl.BlockSpec((1, lanes), lambda i: (0, 0)),
        ],
        out_specs=pl.BlockSpec((None, s2, lanes), lambda i: (i, 0, 0)),
        scratch_shapes=[
            pltpu.VMEM((r0, lanes), jnp.float32),
            pltpu.VMEM((s1, lanes), jnp.bfloat16),
            pltpu.VMEM((r1, lanes), jnp.float32),
        ],
        compiler_params=pltpu.CompilerParams(
            dimension_semantics=("parallel",)),
    )(xf3, w0bd, jnp.asarray(pool0, jnp.bfloat16),
      jnp.asarray(mask0), g0, bt0, w1bd,
      jnp.asarray(pool1, jnp.bfloat16), jnp.asarray(mask1), g1, bt1)

    nc = int(cls_w.shape[0])
    ncp = 128
    wc = cls_w.reshape(nc, cw, s2).transpose(2, 1, 0).reshape(s2 * cw, nc)
    wc = jnp.pad(wc, ((0, 0), (0, ncp - nc)))
    bc = jnp.pad(cls_b, (0, ncp - nc)).reshape(1, ncp)
    flat = y.reshape(ng, s2, grp, cw).transpose(0, 2, 1, 3).reshape(n, s2 * cw)
    logits = pl.pallas_call(
        _linear_kernel,
        out_shape=jax.ShapeDtypeStruct((n, ncp), jnp.float32),
        in_specs=[pl.BlockSpec(memory_space=pltpu.MemorySpace.VMEM)] * 3,
        out_specs=pl.BlockSpec(memory_space=pltpu.MemorySpace.VMEM),
    )(flat, wc, bc)
    return logits[:, :nc]


# speedup vs baseline: 12.0127x; 1.1173x over previous
import functools

import numpy as np

import jax
import jax.numpy as jnp
from jax import lax
from jax.experimental import pallas as pl
from jax.experimental.pallas import tpu as pltpu


def _stage_geom(d, h, w):
    dp, hp, wp = d + 2, h + 2, w + 2
    hwp = hp * wp
    r = (d - 1) * hwp + (h - 1) * wp + (w - 1) + 1
    base = (np.arange(d)[:, None, None] * hwp
            + np.arange(h)[None, :, None] * wp
            + np.arange(w)[None, None, :])
    mask = np.zeros((1, r), np.float32)
    mask[0, base.reshape(-1)] = 1.0 / (d * h * w)
    return dp, hp, wp, r, base, mask


def _pool_mat(base, r, dq, hq, wq, dst_of):
    n_dst = int(np.max(dst_of)) + 1
    p = np.zeros((n_dst, r), np.float32)
    rows = dst_of.reshape(-1)
    for od in range(2):
        for oh in range(2):
            for ow in range(2):
                src = base[od:2 * dq:2, oh:2 * hq:2, ow:2 * wq:2].reshape(-1)
                p[rows, src] = 0.125
    return p


def _net_kernel(xf_ref, w0_ref, p0_ref, m0_ref, g0_ref, b0_ref,
                w1_ref, p1_ref, m1_ref, g1_ref, b1_ref,
                o_ref, xf3_ref, acc0_ref, x1_ref, acc1_ref,
                *, d0, r0, d1, r1, s0, k1):
    for kw in range(3):
        xf3_ref[:, kw * k1:(kw + 1) * k1] = xf_ref[kw:kw + s0, :]
    for t, dl in enumerate(d0):
        part = jnp.dot(xf3_ref[pl.ds(dl, r0), :], w0_ref[t],
                       preferred_element_type=jnp.float32)
        if t == 0:
            acc0_ref[...] = part
        else:
            acc0_ref[...] += part
    acc = acc0_ref[...]
    mean = jnp.dot(m0_ref[...], acc, preferred_element_type=jnp.float32)
    ex2 = jnp.dot(m0_ref[...], acc * acc, preferred_element_type=jnp.float32)
    var = jnp.maximum(ex2 - mean * mean, 0.0)
    scale = g0_ref[...] * lax.rsqrt(var + 1e-5)
    shift = b0_ref[...] - mean * scale
    y = jnp.maximum(acc * scale + shift, 0.0).astype(jnp.bfloat16)
    x1_ref[...] = jnp.dot(p0_ref[...], y,
                          preferred_element_type=jnp.float32).astype(jnp.bfloat16)

    for t, dl in enumerate(d1):
        part = jnp.dot(x1_ref[pl.ds(dl, r1), :], w1_ref[t],
                       preferred_element_type=jnp.float32)
        if t == 0:
            acc1_ref[...] = part
        else:
            acc1_ref[...] += part
    acc1 = acc1_ref[...]
    mean1 = jnp.dot(m1_ref[...], acc1, preferred_element_type=jnp.float32)
    ex21 = jnp.dot(m1_ref[...], acc1 * acc1, preferred_element_type=jnp.float32)
    var1 = jnp.maximum(ex21 - mean1 * mean1, 0.0)
    scale1 = g1_ref[...] * lax.rsqrt(var1 + 1e-5)
    shift1 = b1_ref[...] - mean1 * scale1
    y1 = jnp.maximum(acc1 * scale1 + shift1, 0.0).astype(jnp.bfloat16)
    o_ref[...] = jnp.dot(p1_ref[...], y1, preferred_element_type=jnp.float32)


def _linear_kernel(a_ref, w_ref, b_ref, o_ref):
    o_ref[...] = (jnp.dot(a_ref[...], w_ref[...],
                          preferred_element_type=jnp.float32) + b_ref[...])


def kernel(x, l0_w, l0_b, l0_gamma, l0_beta,
           l1_w, l1_b, l1_gamma, l1_beta, cls_w, cls_b):
    n, c_in, d, h, w = (int(s) for s in x.shape)
    cw = int(l0_w.shape[0])
    grp = 16
    ng = n // grp
    lanes = grp * cw

    dp0, hp0, wp0, r0, base0, mask0 = _stage_geom(d, h, w)
    s0 = dp0 * hp0 * wp0
    d1, h1, w1 = d // 2, h // 2, w // 2
    dp1, hp1, wp1, r1, base1, mask1 = _stage_geom(d1, h1, w1)
    s1 = dp1 * hp1 * wp1
    d2, h2, w2 = d1 // 2, h1 // 2, w1 // 2
    s2 = d2 * h2 * w2

    deltas0 = tuple(kd * (hp0 * wp0) + kh * wp0
                    for kd in range(3) for kh in range(3))
    deltas1 = tuple(kd * (hp1 * wp1) + kh * wp1 + kw
                    for kd in range(3) for kh in range(3) for kw in range(3))

    dst0 = ((np.arange(d1) + 1)[:, None, None] * (hp1 * wp1)
            + (np.arange(h1) + 1)[None, :, None] * wp1
            + (np.arange(w1) + 1)[None, None, :])
    pool0 = np.zeros((s1, r0), np.float32)
    pool0[:dst0.max() + 1] = _pool_mat(base0, r0, d1, h1, w1, dst0)
    dst1 = (np.arange(d2)[:, None, None] * (h2 * w2)
            + np.arange(h2)[None, :, None] * w2
            + np.arange(w2)[None, None, :])
    pool1 = _pool_mat(base1, r1, d2, h2, w2, dst1)

    xb = x.astype(jnp.bfloat16).reshape(ng, grp, c_in, d, h, w)
    xp = jnp.transpose(xb, (0, 3, 4, 5, 1, 2))
    xp = jnp.pad(xp, ((0, 0), (1, 1), (1, 1), (1, 1), (0, 0), (0, 0)))
    xf = xp.reshape(ng, s0, grp * c_in)
    xf = jnp.pad(xf, ((0, 0), (0, 2), (0, 0)))

    eye = jnp.eye(grp, dtype=jnp.float32)
    w0t = jnp.transpose(l0_w, (2, 3, 4, 1, 0)).reshape(9, 3, c_in, cw)
    w0bd = jnp.einsum('gh,tkio->tkgiho', eye, w0t)
    w0bd = w0bd.reshape(9, 3 * grp * c_in, lanes).astype(jnp.bfloat16)
    w1t = jnp.transpose(l1_w, (2, 3, 4, 1, 0)).reshape(27, cw, cw)
    w1bd = jnp.einsum('gh,tio->tgiho', eye, w1t)
    w1bd = w1bd.reshape(27, grp * cw, lanes).astype(jnp.bfloat16)

    g0 = jnp.tile(l0_gamma, grp).reshape(1, lanes)
    bt0 = jnp.tile(l0_beta, grp).reshape(1, lanes)
    g1 = jnp.tile(l1_gamma, grp).reshape(1, lanes)
    bt1 = jnp.tile(l1_beta, grp).reshape(1, lanes)

    k_in = 3 * grp * c_in
    _body = functools.partial(_net_kernel, d0=deltas0, r0=r0,
                              d1=deltas1, r1=r1, s0=s0, k1=grp * c_in)
    y = pl.pallas_call(
        _body,
        out_shape=jax.ShapeDtypeStruct((ng, s2, lanes), jnp.float32),
        grid=(ng,),
        in_specs=[
            pl.BlockSpec((None, s0 + 2, grp * c_in), lambda i: (i, 0, 0)),
            pl.BlockSpec((9, k_in, lanes), lambda i: (0, 0, 0)),
            pl.BlockSpec((s1, r0), lambda i: (0, 0)),
            pl.BlockSpec((1, r0), lambda i: (0, 0)),
            pl.BlockSpec((1, lanes), lambda i: (0, 0)),
            pl.BlockSpec((1, lanes), lambda i: (0, 0)),
            pl.BlockSpec((27, grp * cw, lanes), lambda i: (0, 0, 0)),
            pl.BlockSpec((s2, r1), lambda i: (0, 0)),
            pl.BlockSpec((1, r1), lambda i: (0, 0)),
            pl.BlockSpec((1, lanes), lambda i: (0, 0)),
            pl.BlockSpec((1, lanes), lambda i: (0, 0)),
        ],
        out_specs=pl.BlockSpec((None, s2, lanes), lambda i: (i, 0, 0)),
        scratch_shapes=[
            pltpu.VMEM((s0, k_in), jnp.bfloat16),
            pltpu.VMEM((r0, lanes), jnp.float32),
            pltpu.VMEM((s1, lanes), jnp.bfloat16),
            pltpu.VMEM((r1, lanes), jnp.float32),
        ],
        compiler_params=pltpu.CompilerParams(
            dimension_semantics=("parallel",)),
    )(xf, w0bd, jnp.asarray(pool0, jnp.bfloat16),
      jnp.asarray(mask0), g0, bt0, w1bd,
      jnp.asarray(pool1, jnp.bfloat16), jnp.asarray(mask1), g1, bt1)

    nc = int(cls_w.shape[0])
    ncp = 128
    wc = cls_w.reshape(nc, cw, s2).transpose(2, 1, 0).reshape(s2 * cw, nc)
    wc = jnp.pad(wc, ((0, 0), (0, ncp - nc)))
    bc = jnp.pad(cls_b, (0, ncp - nc)).reshape(1, ncp)
    flat = y.reshape(ng, s2, grp, cw).transpose(0, 2, 1, 3).reshape(n, s2 * cw)
    logits = pl.pallas_call(
        _linear_kernel,
        out_shape=jax.ShapeDtypeStruct((n, ncp), jnp.float32),
        in_specs=[pl.BlockSpec(memory_space=pltpu.MemorySpace.VMEM)] * 3,
        out_specs=pl.BlockSpec(memory_space=pltpu.MemorySpace.VMEM),
    )(flat, wc, bc)
    return logits[:, :nc]


# speedup vs baseline: 13.8724x; 1.1548x over previous
import functools

import numpy as np

import jax
import jax.numpy as jnp
from jax import lax
from jax.experimental import pallas as pl
from jax.experimental.pallas import tpu as pltpu


def _stage_geom(d, h, w):
    dp, hp, wp = d + 2, h + 2, w + 2
    hwp = hp * wp
    r = (d - 1) * hwp + (h - 1) * wp + (w - 1) + 1
    base = (np.arange(d)[:, None, None] * hwp
            + np.arange(h)[None, :, None] * wp
            + np.arange(w)[None, None, :])
    mask = np.zeros((1, r), np.float32)
    mask[0, base.reshape(-1)] = 1.0 / (d * h * w)
    return dp, hp, wp, r, base, mask


def _pool_mat(base, r, dq, hq, wq, dst_of):
    n_dst = int(np.max(dst_of)) + 1
    p = np.zeros((n_dst, r), np.float32)
    rows = dst_of.reshape(-1)
    for od in range(2):
        for oh in range(2):
            for ow in range(2):
                src = base[od:2 * dq:2, oh:2 * hq:2, ow:2 * wq:2].reshape(-1)
                p[rows, src] = 0.125
    return p


def _net_kernel(xf_ref, w0_ref, p0_ref, m0_ref, g0_ref, b0_ref,
                w1_ref, p1_ref, m1_ref, g1_ref, b1_ref,
                o_ref, xf3_ref, acc0_ref, y0_ref, x1_ref, acc1_ref,
                *, d0, r0, d1, r1, s0, k1, hwp0, wp1, hw1, dq1, slab_k):
    for kw in range(3):
        xf3_ref[:, kw * k1:(kw + 1) * k1] = xf_ref[kw:kw + s0, :]
    for t, dl in enumerate(d0):
        part = jnp.dot(xf3_ref[pl.ds(dl, r0), :], w0_ref[t],
                       preferred_element_type=jnp.float32)
        if t == 0:
            acc0_ref[...] = part
        else:
            acc0_ref[...] += part
    acc = acc0_ref[...]
    mean = jnp.dot(m0_ref[...], acc, preferred_element_type=jnp.float32)
    ex2 = jnp.dot(m0_ref[...], acc * acc, preferred_element_type=jnp.float32)
    var = jnp.maximum(ex2 - mean * mean, 0.0)
    scale = g0_ref[...] * lax.rsqrt(var + 1e-5)
    shift = b0_ref[...] - mean * scale
    y0_ref[...] = jnp.maximum(acc * scale + shift, 0.0).astype(jnp.bfloat16)
    zero_slab = jnp.zeros((hw1, x1_ref.shape[1]), jnp.bfloat16)
    x1_ref[0:hw1, :] = zero_slab
    x1_ref[(dq1 + 1) * hw1:(dq1 + 2) * hw1, :] = zero_slab
    for od in range(dq1):
        sl = jnp.dot(p0_ref[...], y0_ref[pl.ds(2 * od * hwp0, slab_k), :],
                     preferred_element_type=jnp.float32)
        x1_ref[pl.ds((od + 1) * hw1, hw1), :] = sl.astype(jnp.bfloat16)

    for t, dl in enumerate(d1):
        part = jnp.dot(x1_ref[pl.ds(dl, r1), :], w1_ref[t],
                       preferred_element_type=jnp.float32)
        if t == 0:
            acc1_ref[...] = part
        else:
            acc1_ref[...] += part
    acc1 = acc1_ref[...]
    mean1 = jnp.dot(m1_ref[...], acc1, preferred_element_type=jnp.float32)
    ex21 = jnp.dot(m1_ref[...], acc1 * acc1, preferred_element_type=jnp.float32)
    var1 = jnp.maximum(ex21 - mean1 * mean1, 0.0)
    scale1 = g1_ref[...] * lax.rsqrt(var1 + 1e-5)
    shift1 = b1_ref[...] - mean1 * scale1
    y1 = jnp.maximum(acc1 * scale1 + shift1, 0.0).astype(jnp.bfloat16)
    o_ref[...] = jnp.dot(p1_ref[...], y1, preferred_element_type=jnp.float32)


def _linear_kernel(a_ref, w_ref, b_ref, o_ref):
    o_ref[...] = (jnp.dot(a_ref[...], w_ref[...],
                          preferred_element_type=jnp.float32) + b_ref[...])


def kernel(x, l0_w, l0_b, l0_gamma, l0_beta,
           l1_w, l1_b, l1_gamma, l1_beta, cls_w, cls_b):
    n, c_in, d, h, w = (int(s) for s in x.shape)
    cw = int(l0_w.shape[0])
    grp = 16
    ng = n // grp
    lanes = grp * cw

    dp0, hp0, wp0, r0, base0, mask0 = _stage_geom(d, h, w)
    s0 = dp0 * hp0 * wp0
    d1, h1, w1 = d // 2, h // 2, w // 2
    dp1, hp1, wp1, r1, base1, mask1 = _stage_geom(d1, h1, w1)
    s1 = dp1 * hp1 * wp1
    d2, h2, w2 = d1 // 2, h1 // 2, w1 // 2
    s2 = d2 * h2 * w2

    deltas0 = tuple(kd * (hp0 * wp0) + kh * wp0
                    for kd in range(3) for kh in range(3))
    deltas1 = tuple(kd * (hp1 * wp1) + kh * wp1 + kw
                    for kd in range(3) for kh in range(3) for kw in range(3))

    hwp0 = hp0 * wp0
    hw1 = hp1 * wp1
    slab_k = hwp0 + (h - 1) * wp0 + (w - 1) + 1
    pool0 = np.zeros((hw1, slab_k), np.float32)
    for i in range(2):
        for j in range(2):
            for k in range(2):
                src = (i * hwp0 + (2 * np.arange(h1)[:, None] + j) * wp0
                       + 2 * np.arange(w1)[None, :] + k)
                dst = ((np.arange(h1)[:, None] + 1) * wp1
                       + np.arange(w1)[None, :] + 1)
                pool0[dst.reshape(-1), src.reshape(-1)] = 0.125
    dst1 = (np.arange(d2)[:, None, None] * (h2 * w2)
            + np.arange(h2)[None, :, None] * w2
            + np.arange(w2)[None, None, :])
    pool1 = _pool_mat(base1, r1, d2, h2, w2, dst1)

    xb = x.astype(jnp.bfloat16).reshape(ng, grp, c_in, d, h, w)
    xp = jnp.transpose(xb, (0, 3, 4, 5, 1, 2))
    xp = jnp.pad(xp, ((0, 0), (1, 1), (1, 1), (1, 1), (0, 0), (0, 0)))
    k1 = 64
    xf = xp.reshape(ng, s0, grp * c_in)
    xf = jnp.pad(xf, ((0, 0), (0, 2), (0, k1 - grp * c_in)))

    eye = jnp.eye(grp, dtype=jnp.float32)
    w0t = jnp.transpose(l0_w, (2, 3, 4, 1, 0)).reshape(9, 3, c_in, cw)
    w0bd = jnp.einsum('gh,tkio->tkgiho', eye, w0t)
    w0bd = w0bd.reshape(9, 3, grp * c_in, lanes)
    w0bd = jnp.pad(w0bd, ((0, 0), (0, 0), (0, k1 - grp * c_in), (0, 0)))
    w0bd = w0bd.reshape(9, 3 * k1, lanes).astype(jnp.bfloat16)
    w1t = jnp.transpose(l1_w, (2, 3, 4, 1, 0)).reshape(27, cw, cw)
    w1bd = jnp.einsum('gh,tio->tgiho', eye, w1t)
    w1bd = w1bd.reshape(27, grp * cw, lanes).astype(jnp.bfloat16)

    g0 = jnp.tile(l0_gamma, grp).reshape(1, lanes)
    bt0 = jnp.tile(l0_beta, grp).reshape(1, lanes)
    g1 = jnp.tile(l1_gamma, grp).reshape(1, lanes)
    bt1 = jnp.tile(l1_beta, grp).reshape(1, lanes)

    k_in = 3 * k1
    _body = functools.partial(_net_kernel, d0=deltas0, r0=r0,
                              d1=deltas1, r1=r1, s0=s0, k1=k1,
                              hwp0=hwp0, wp1=wp1, hw1=hw1, dq1=d1,
                              slab_k=slab_k)
    y = pl.pallas_call(
        _body,
        out_shape=jax.ShapeDtypeStruct((ng, s2, lanes), jnp.float32),
        grid=(ng,),
        in_specs=[
            pl.BlockSpec((None, s0 + 2, k1), lambda i: (i, 0, 0)),
            pl.BlockSpec((9, k_in, lanes), lambda i: (0, 0, 0)),
            pl.BlockSpec((hw1, slab_k), lambda i: (0, 0)),
            pl.BlockSpec((1, r0), lambda i: (0, 0)),
            pl.BlockSpec((1, lanes), lambda i: (0, 0)),
            pl.BlockSpec((1, lanes), lambda i: (0, 0)),
            pl.BlockSpec((27, grp * cw, lanes), lambda i: (0, 0, 0)),
            pl.BlockSpec((s2, r1), lambda i: (0, 0)),
            pl.BlockSpec((1, r1), lambda i: (0, 0)),
            pl.BlockSpec((1, lanes), lambda i: (0, 0)),
            pl.BlockSpec((1, lanes), lambda i: (0, 0)),
        ],
        out_specs=pl.BlockSpec((None, s2, lanes), lambda i: (i, 0, 0)),
        scratch_shapes=[
            pltpu.VMEM((s0, k_in), jnp.bfloat16),
            pltpu.VMEM((r0, lanes), jnp.float32),
            pltpu.VMEM((r0, lanes), jnp.bfloat16),
            pltpu.VMEM((s1, lanes), jnp.bfloat16),
            pltpu.VMEM((r1, lanes), jnp.float32),
        ],
        compiler_params=pltpu.CompilerParams(
            dimension_semantics=("parallel",)),
    )(xf, w0bd, jnp.asarray(pool0, jnp.bfloat16),
      jnp.asarray(mask0), g0, bt0, w1bd,
      jnp.asarray(pool1, jnp.bfloat16), jnp.asarray(mask1), g1, bt1)

    nc = int(cls_w.shape[0])
    ncp = 128
    wc = cls_w.reshape(nc, cw, s2).transpose(2, 1, 0).reshape(s2 * cw, nc)
    wc = jnp.pad(wc, ((0, 0), (0, ncp - nc)))
    bc = jnp.pad(cls_b, (0, ncp - nc)).reshape(1, ncp)
    flat = y.reshape(ng, s2, grp, cw).transpose(0, 2, 1, 3).reshape(n, s2 * cw)
    logits = pl.pallas_call(
        _linear_kernel,
        out_shape=jax.ShapeDtypeStruct((n, ncp), jnp.float32),
        in_specs=[pl.BlockSpec(memory_space=pltpu.MemorySpace.VMEM)] * 3,
        out_specs=pl.BlockSpec(memory_space=pltpu.MemorySpace.VMEM),
    )(flat, wc, bc)
    return logits[:, :nc]


# speedup vs baseline: 16.7557x; 1.2078x over previous
import functools

import numpy as np

import jax
import jax.numpy as jnp
from jax import lax
from jax.experimental import pallas as pl
from jax.experimental.pallas import tpu as pltpu


def _stage_geom(d, h, w):
    dp, hp, wp = d + 2, h + 2, w + 2
    hwp = hp * wp
    r = (d - 1) * hwp + (h - 1) * wp + (w - 1) + 1
    base = (np.arange(d)[:, None, None] * hwp
            + np.arange(h)[None, :, None] * wp
            + np.arange(w)[None, None, :])
    mask = np.zeros((1, r), np.float32)
    mask[0, base.reshape(-1)] = 1.0 / (d * h * w)
    return dp, hp, wp, r, base, mask


def _pool_mat(base, r, dq, hq, wq, dst_of):
    n_dst = int(np.max(dst_of)) + 1
    p = np.zeros((n_dst, r), np.float32)
    rows = dst_of.reshape(-1)
    for od in range(2):
        for oh in range(2):
            for ow in range(2):
                src = base[od:2 * dq:2, oh:2 * hq:2, ow:2 * wq:2].reshape(-1)
                p[rows, src] = 0.125
    return p


def _net_kernel(xf_ref, w0_ref, p0_ref, m0_ref, g0_ref, b0_ref,
                w1_ref, p1_ref, m1_ref, g1_ref, b1_ref,
                o_ref, xf3_ref, acc0_ref, y0_ref, x1_ref, acc1_ref,
                *, d0, r0, d1, r1, s0, k1, hwp0, wp1, hw1, dq1, slab_k,
                nd0, m0rows, nd1, m1rows):
    for kw in range(3):
        xf3_ref[:, kw * k1:(kw + 1) * k1] = xf_ref[kw:kw + s0, :]
    for ds in range(nd0):
        base = ds * hwp0
        tot = None
        for t, dl in enumerate(d0):
            p = jnp.dot(xf3_ref[pl.ds(base + dl, m0rows), :], w0_ref[t],
                        preferred_element_type=jnp.float32)
            tot = p if tot is None else tot + p
        acc0_ref[pl.ds(base, m0rows), :] = tot
        if ds < nd0 - 1:
            acc0_ref[pl.ds(base + m0rows, hwp0 - m0rows), :] = (
                jnp.zeros((hwp0 - m0rows, tot.shape[1]), jnp.float32))
    acc = acc0_ref[...]
    mean = jnp.dot(m0_ref[...], acc, preferred_element_type=jnp.float32)
    ex2 = jnp.dot(m0_ref[...], acc * acc, preferred_element_type=jnp.float32)
    var = jnp.maximum(ex2 - mean * mean, 0.0)
    scale = g0_ref[...] * lax.rsqrt(var + 1e-5)
    shift = b0_ref[...] - mean * scale
    y0_ref[...] = jnp.maximum(acc * scale + shift, 0.0).astype(jnp.bfloat16)
    zero_slab = jnp.zeros((hw1, x1_ref.shape[1]), jnp.bfloat16)
    x1_ref[0:hw1, :] = zero_slab
    x1_ref[(dq1 + 1) * hw1:(dq1 + 2) * hw1, :] = zero_slab
    for od in range(dq1):
        sl = jnp.dot(p0_ref[...], y0_ref[pl.ds(2 * od * hwp0, slab_k), :],
                     preferred_element_type=jnp.float32)
        x1_ref[pl.ds((od + 1) * hw1, hw1), :] = sl.astype(jnp.bfloat16)

    for ds in range(nd1):
        base = ds * hw1
        tot = None
        for t, dl in enumerate(d1):
            p = jnp.dot(x1_ref[pl.ds(base + dl, m1rows), :], w1_ref[t],
                        preferred_element_type=jnp.float32)
            tot = p if tot is None else tot + p
        acc1_ref[pl.ds(base, m1rows), :] = tot
        if ds < nd1 - 1:
            acc1_ref[pl.ds(base + m1rows, hw1 - m1rows), :] = (
                jnp.zeros((hw1 - m1rows, tot.shape[1]), jnp.float32))
    acc1 = acc1_ref[...]
    mean1 = jnp.dot(m1_ref[...], acc1, preferred_element_type=jnp.float32)
    ex21 = jnp.dot(m1_ref[...], acc1 * acc1, preferred_element_type=jnp.float32)
    var1 = jnp.maximum(ex21 - mean1 * mean1, 0.0)
    scale1 = g1_ref[...] * lax.rsqrt(var1 + 1e-5)
    shift1 = b1_ref[...] - mean1 * scale1
    y1 = jnp.maximum(acc1 * scale1 + shift1, 0.0).astype(jnp.bfloat16)
    o_ref[...] = jnp.dot(p1_ref[...], y1, preferred_element_type=jnp.float32)


def _linear_kernel(a_ref, w_ref, b_ref, o_ref):
    o_ref[...] = (jnp.dot(a_ref[...], w_ref[...],
                          preferred_element_type=jnp.float32) + b_ref[...])


def kernel(x, l0_w, l0_b, l0_gamma, l0_beta,
           l1_w, l1_b, l1_gamma, l1_beta, cls_w, cls_b):
    n, c_in, d, h, w = (int(s) for s in x.shape)
    cw = int(l0_w.shape[0])
    grp = 16
    ng = n // grp
    lanes = grp * cw

    dp0, hp0, wp0, r0, base0, mask0 = _stage_geom(d, h, w)
    s0 = dp0 * hp0 * wp0
    d1, h1, w1 = d // 2, h // 2, w // 2
    dp1, hp1, wp1, r1, base1, mask1 = _stage_geom(d1, h1, w1)
    s1 = dp1 * hp1 * wp1
    d2, h2, w2 = d1 // 2, h1 // 2, w1 // 2
    s2 = d2 * h2 * w2

    deltas0 = tuple(kd * (hp0 * wp0) + kh * wp0
                    for kd in range(3) for kh in range(3))
    deltas1 = tuple(kd * (hp1 * wp1) + kh * wp1 + kw
                    for kd in range(3) for kh in range(3) for kw in range(3))

    hwp0 = hp0 * wp0
    hw1 = hp1 * wp1
    slab_k = hwp0 + (h - 1) * wp0 + (w - 1) + 1
    pool0 = np.zeros((hw1, slab_k), np.float32)
    for i in range(2):
        for j in range(2):
            for k in range(2):
                src = (i * hwp0 + (2 * np.arange(h1)[:, None] + j) * wp0
                       + 2 * np.arange(w1)[None, :] + k)
                dst = ((np.arange(h1)[:, None] + 1) * wp1
                       + np.arange(w1)[None, :] + 1)
                pool0[dst.reshape(-1), src.reshape(-1)] = 0.125
    dst1 = (np.arange(d2)[:, None, None] * (h2 * w2)
            + np.arange(h2)[None, :, None] * w2
            + np.arange(w2)[None, None, :])
    pool1 = _pool_mat(base1, r1, d2, h2, w2, dst1)

    xb = x.astype(jnp.bfloat16).reshape(ng, grp, c_in, d, h, w)
    xp = jnp.transpose(xb, (0, 3, 4, 5, 1, 2))
    xp = jnp.pad(xp, ((0, 0), (1, 1), (1, 1), (1, 1), (0, 0), (0, 0)))
    k1 = 64
    xf = xp.reshape(ng, s0, grp * c_in)
    xf = jnp.pad(xf, ((0, 0), (0, 2), (0, k1 - grp * c_in)))

    eye = jnp.eye(grp, dtype=jnp.float32)
    w0t = jnp.transpose(l0_w, (2, 3, 4, 1, 0)).reshape(9, 3, c_in, cw)
    w0bd = jnp.einsum('gh,tkio->tkgiho', eye, w0t)
    w0bd = w0bd.reshape(9, 3, grp * c_in, lanes)
    w0bd = jnp.pad(w0bd, ((0, 0), (0, 0), (0, k1 - grp * c_in), (0, 0)))
    w0bd = w0bd.reshape(9, 3 * k1, lanes).astype(jnp.bfloat16)
    w1t = jnp.transpose(l1_w, (2, 3, 4, 1, 0)).reshape(27, cw, cw)
    w1bd = jnp.einsum('gh,tio->tgiho', eye, w1t)
    w1bd = w1bd.reshape(27, grp * cw, lanes).astype(jnp.bfloat16)

    g0 = jnp.tile(l0_gamma, grp).reshape(1, lanes)
    bt0 = jnp.tile(l0_beta, grp).reshape(1, lanes)
    g1 = jnp.tile(l1_gamma, grp).reshape(1, lanes)
    bt1 = jnp.tile(l1_beta, grp).reshape(1, lanes)

    k_in = 3 * k1
    _body = functools.partial(_net_kernel, d0=deltas0, r0=r0,
                              d1=deltas1, r1=r1, s0=s0, k1=k1,
                              hwp0=hwp0, wp1=wp1, hw1=hw1, dq1=d1,
                              slab_k=slab_k,
                              nd0=d, m0rows=(h - 1) * wp0 + w,
                              nd1=d1, m1rows=(h1 - 1) * wp1 + w1)
    y = pl.pallas_call(
        _body,
        out_shape=jax.ShapeDtypeStruct((ng, s2, lanes), jnp.float32),
        grid=(ng,),
        in_specs=[
            pl.BlockSpec((None, s0 + 2, k1), lambda i: (i, 0, 0)),
            pl.BlockSpec((9, k_in, lanes), lambda i: (0, 0, 0)),
            pl.BlockSpec((hw1, slab_k), lambda i: (0, 0)),
            pl.BlockSpec((1, r0), lambda i: (0, 0)),
            pl.BlockSpec((1, lanes), lambda i: (0, 0)),
            pl.BlockSpec((1, lanes), lambda i: (0, 0)),
            pl.BlockSpec((27, grp * cw, lanes), lambda i: (0, 0, 0)),
            pl.BlockSpec((s2, r1), lambda i: (0, 0)),
            pl.BlockSpec((1, r1), lambda i: (0, 0)),
            pl.BlockSpec((1, lanes), lambda i: (0, 0)),
            pl.BlockSpec((1, lanes), lambda i: (0, 0)),
        ],
        out_specs=pl.BlockSpec((None, s2, lanes), lambda i: (i, 0, 0)),
        scratch_shapes=[
            pltpu.VMEM((s0, k_in), jnp.bfloat16),
            pltpu.VMEM((r0, lanes), jnp.float32),
            pltpu.VMEM((r0, lanes), jnp.bfloat16),
            pltpu.VMEM((s1, lanes), jnp.bfloat16),
            pltpu.VMEM((r1, lanes), jnp.float32),
        ],
        compiler_params=pltpu.CompilerParams(
            dimension_semantics=("parallel",)),
    )(xf, w0bd, jnp.asarray(pool0, jnp.bfloat16),
      jnp.asarray(mask0), g0, bt0, w1bd,
      jnp.asarray(pool1, jnp.bfloat16), jnp.asarray(mask1), g1, bt1)

    nc = int(cls_w.shape[0])
    ncp = 128
    wc = cls_w.reshape(nc, cw, s2).transpose(2, 1, 0).reshape(s2 * cw, nc)
    wc = jnp.pad(wc, ((0, 0), (0, ncp - nc)))
    bc = jnp.pad(cls_b, (0, ncp - nc)).reshape(1, ncp)
    flat = y.reshape(ng, s2, grp, cw).transpose(0, 2, 1, 3).reshape(n, s2 * cw)
    logits = pl.pallas_call(
        _linear_kernel,
        out_shape=jax.ShapeDtypeStruct((n, ncp), jnp.float32),
        in_specs=[pl.BlockSpec(memory_space=pltpu.MemorySpace.VMEM)] * 3,
        out_specs=pl.BlockSpec(memory_space=pltpu.MemorySpace.VMEM),
    )(flat, wc, bc)
    return logits[:, :nc]


# speedup vs baseline: 21.1140x; 1.2601x over previous
import functools

import numpy as np

import jax
import jax.numpy as jnp
from jax import lax
from jax.experimental import pallas as pl
from jax.experimental.pallas import tpu as pltpu


def _stage_geom(d, h, w):
    dp, hp, wp = d + 2, h + 2, w + 2
    hwp = hp * wp
    r = (d - 1) * hwp + (h - 1) * wp + (w - 1) + 1
    base = (np.arange(d)[:, None, None] * hwp
            + np.arange(h)[None, :, None] * wp
            + np.arange(w)[None, None, :])
    mask = np.zeros((1, r), np.float32)
    mask[0, base.reshape(-1)] = 1.0 / (d * h * w)
    return dp, hp, wp, r, base, mask


def _pool_mat(base, r, dq, hq, wq, dst_of):
    n_dst = int(np.max(dst_of)) + 1
    p = np.zeros((n_dst, r), np.float32)
    rows = dst_of.reshape(-1)
    for od in range(2):
        for oh in range(2):
            for ow in range(2):
                src = base[od:2 * dq:2, oh:2 * hq:2, ow:2 * wq:2].reshape(-1)
                p[rows, src] = 0.125
    return p


def _net_kernel(xf_ref, w0_ref, p0_ref, m0_ref, g0_ref, b0_ref,
                w1_ref, p1_ref, m1_ref, g1_ref, b1_ref,
                o_ref, xf3_ref, acc0_ref, y0_ref, x1_ref, acc1_ref,
                *, d0, r0, d1, r1, s0, k1, hwp0, wp1, hw1, dq1, slab_k,
                nd0, m0rows, nd1, m1rows):
    xt = jnp.transpose(xf_ref[...], (1, 0))
    xt = jnp.pad(xt, ((0, 0), (0, k1 - xt.shape[1])))
    for kw in range(3):
        xf3_ref[0:s0 - kw, kw * k1:(kw + 1) * k1] = xt[kw:s0, :]
    for ds in range(nd0):
        base = ds * hwp0
        tot = None
        for t, dl in enumerate(d0):
            p = jnp.dot(xf3_ref[pl.ds(base + dl, m0rows), :], w0_ref[t],
                        preferred_element_type=jnp.float32)
            tot = p if tot is None else tot + p
        acc0_ref[pl.ds(base, m0rows), :] = tot
        if ds < nd0 - 1:
            acc0_ref[pl.ds(base + m0rows, hwp0 - m0rows), :] = (
                jnp.zeros((hwp0 - m0rows, tot.shape[1]), jnp.float32))
    acc = acc0_ref[...]
    mean = jnp.dot(m0_ref[...], acc, preferred_element_type=jnp.float32)
    ex2 = jnp.dot(m0_ref[...], acc * acc, preferred_element_type=jnp.float32)
    var = jnp.maximum(ex2 - mean * mean, 0.0)
    scale = g0_ref[...] * lax.rsqrt(var + 1e-5)
    shift = b0_ref[...] - mean * scale
    y0_ref[...] = jnp.maximum(acc * scale + shift, 0.0).astype(jnp.bfloat16)
    zero_slab = jnp.zeros((hw1, x1_ref.shape[1]), jnp.bfloat16)
    x1_ref[0:hw1, :] = zero_slab
    x1_ref[(dq1 + 1) * hw1:(dq1 + 2) * hw1, :] = zero_slab
    for od in range(dq1):
        sl = jnp.dot(p0_ref[...], y0_ref[pl.ds(2 * od * hwp0, slab_k), :],
                     preferred_element_type=jnp.float32)
        x1_ref[pl.ds((od + 1) * hw1, hw1), :] = sl.astype(jnp.bfloat16)

    for ds in range(nd1):
        base = ds * hw1
        tot = None
        for t, dl in enumerate(d1):
            p = jnp.dot(x1_ref[pl.ds(base + dl, m1rows), :], w1_ref[t],
                        preferred_element_type=jnp.float32)
            tot = p if tot is None else tot + p
        acc1_ref[pl.ds(base, m1rows), :] = tot
        if ds < nd1 - 1:
            acc1_ref[pl.ds(base + m1rows, hw1 - m1rows), :] = (
                jnp.zeros((hw1 - m1rows, tot.shape[1]), jnp.float32))
    acc1 = acc1_ref[...]
    mean1 = jnp.dot(m1_ref[...], acc1, preferred_element_type=jnp.float32)
    ex21 = jnp.dot(m1_ref[...], acc1 * acc1, preferred_element_type=jnp.float32)
    var1 = jnp.maximum(ex21 - mean1 * mean1, 0.0)
    scale1 = g1_ref[...] * lax.rsqrt(var1 + 1e-5)
    shift1 = b1_ref[...] - mean1 * scale1
    y1 = jnp.maximum(acc1 * scale1 + shift1, 0.0).astype(jnp.bfloat16)
    o_ref[...] = jnp.dot(p1_ref[...], y1, preferred_element_type=jnp.float32)


def _linear_kernel(a_ref, w_ref, b_ref, o_ref):
    o_ref[...] = (jnp.dot(a_ref[...], w_ref[...],
                          preferred_element_type=jnp.float32) + b_ref[...])


def kernel(x, l0_w, l0_b, l0_gamma, l0_beta,
           l1_w, l1_b, l1_gamma, l1_beta, cls_w, cls_b):
    n, c_in, d, h, w = (int(s) for s in x.shape)
    cw = int(l0_w.shape[0])
    grp = 16
    ng = n // grp
    lanes = grp * cw

    dp0, hp0, wp0, r0, base0, mask0 = _stage_geom(d, h, w)
    s0 = dp0 * hp0 * wp0
    d1, h1, w1 = d // 2, h // 2, w // 2
    dp1, hp1, wp1, r1, base1, mask1 = _stage_geom(d1, h1, w1)
    s1 = dp1 * hp1 * wp1
    d2, h2, w2 = d1 // 2, h1 // 2, w1 // 2
    s2 = d2 * h2 * w2

    deltas0 = tuple(kd * (hp0 * wp0) + kh * wp0
                    for kd in range(3) for kh in range(3))
    deltas1 = tuple(kd * (hp1 * wp1) + kh * wp1 + kw
                    for kd in range(3) for kh in range(3) for kw in range(3))

    hwp0 = hp0 * wp0
    hw1 = hp1 * wp1
    slab_k = hwp0 + (h - 1) * wp0 + (w - 1) + 1
    pool0 = np.zeros((hw1, slab_k), np.float32)
    for i in range(2):
        for j in range(2):
            for k in range(2):
                src = (i * hwp0 + (2 * np.arange(h1)[:, None] + j) * wp0
                       + 2 * np.arange(w1)[None, :] + k)
                dst = ((np.arange(h1)[:, None] + 1) * wp1
                       + np.arange(w1)[None, :] + 1)
                pool0[dst.reshape(-1), src.reshape(-1)] = 0.125
    dst1 = (np.arange(d2)[:, None, None] * (h2 * w2)
            + np.arange(h2)[None, :, None] * w2
            + np.arange(w2)[None, None, :])
    pool1 = _pool_mat(base1, r1, d2, h2, w2, dst1)

    k1 = 64
    xp = jnp.pad(x.astype(jnp.bfloat16),
                 ((0, 0), (0, 0), (1, 1), (1, 1), (1, 1)))
    xf = xp.reshape(ng, grp * c_in, s0)

    eye = jnp.eye(grp, dtype=jnp.float32)
    w0t = jnp.transpose(l0_w, (2, 3, 4, 1, 0)).reshape(9, 3, c_in, cw)
    w0bd = jnp.einsum('gh,tkio->tkgiho', eye, w0t)
    w0bd = w0bd.reshape(9, 3, grp * c_in, lanes)
    w0bd = jnp.pad(w0bd, ((0, 0), (0, 0), (0, k1 - grp * c_in), (0, 0)))
    w0bd = w0bd.reshape(9, 3 * k1, lanes).astype(jnp.bfloat16)
    w1t = jnp.transpose(l1_w, (2, 3, 4, 1, 0)).reshape(27, cw, cw)
    w1bd = jnp.einsum('gh,tio->tgiho', eye, w1t)
    w1bd = w1bd.reshape(27, grp * cw, lanes).astype(jnp.bfloat16)

    g0 = jnp.tile(l0_gamma, grp).reshape(1, lanes)
    bt0 = jnp.tile(l0_beta, grp).reshape(1, lanes)
    g1 = jnp.tile(l1_gamma, grp).reshape(1, lanes)
    bt1 = jnp.tile(l1_beta, grp).reshape(1, lanes)

    k_in = 3 * k1
    _body = functools.partial(_net_kernel, d0=deltas0, r0=r0,
                              d1=deltas1, r1=r1, s0=s0, k1=k1,
                              hwp0=hwp0, wp1=wp1, hw1=hw1, dq1=d1,
                              slab_k=slab_k,
                              nd0=d, m0rows=(h - 1) * wp0 + w,
                              nd1=d1, m1rows=(h1 - 1) * wp1 + w1)
    y = pl.pallas_call(
        _body,
        out_shape=jax.ShapeDtypeStruct((ng, s2, lanes), jnp.float32),
        grid=(ng,),
        in_specs=[
            pl.BlockSpec((None, grp * c_in, s0), lambda i: (i, 0, 0)),
            pl.BlockSpec((9, k_in, lanes), lambda i: (0, 0, 0)),
            pl.BlockSpec((hw1, slab_k), lambda i: (0, 0)),
            pl.BlockSpec((1, r0), lambda i: (0, 0)),
            pl.BlockSpec((1, lanes), lambda i: (0, 0)),
            pl.BlockSpec((1, lanes), lambda i: (0, 0)),
            pl.BlockSpec((27, grp * cw, lanes), lambda i: (0, 0, 0)),
            pl.BlockSpec((s2, r1), lambda i: (0, 0)),
            pl.BlockSpec((1, r1), lambda i: (0, 0)),
            pl.BlockSpec((1, lanes), lambda i: (0, 0)),
            pl.BlockSpec((1, lanes), lambda i: (0, 0)),
        ],
        out_specs=pl.BlockSpec((None, s2, lanes), lambda i: (i, 0, 0)),
        scratch_shapes=[
            pltpu.VMEM((s0, k_in), jnp.bfloat16),
            pltpu.VMEM((r0, lanes), jnp.float32),
            pltpu.VMEM((r0, lanes), jnp.bfloat16),
            pltpu.VMEM((s1, lanes), jnp.bfloat16),
            pltpu.VMEM((r1, lanes), jnp.float32),
        ],
        compiler_params=pltpu.CompilerParams(
            dimension_semantics=("parallel",)),
    )(xf, w0bd, jnp.asarray(pool0, jnp.bfloat16),
      jnp.asarray(mask0), g0, bt0, w1bd,
      jnp.asarray(pool1, jnp.bfloat16), jnp.asarray(mask1), g1, bt1)

    nc = int(cls_w.shape[0])
    ncp = 128
    wc = cls_w.reshape(nc, cw, s2).transpose(2, 1, 0).reshape(s2 * cw, nc)
    wc = jnp.pad(wc, ((0, 0), (0, ncp - nc)))
    bc = jnp.pad(cls_b, (0, ncp - nc)).reshape(1, ncp)
    flat = y.reshape(ng, s2, grp, cw).transpose(0, 2, 1, 3).reshape(n, s2 * cw)
    logits = pl.pallas_call(
        _linear_kernel,
        out_shape=jax.ShapeDtypeStruct((n, ncp), jnp.float32),
        in_specs=[pl.BlockSpec(memory_space=pltpu.MemorySpace.VMEM)] * 3,
        out_specs=pl.BlockSpec(memory_space=pltpu.MemorySpace.VMEM),
    )(flat, wc, bc)
    return logits[:, :nc]


# speedup vs baseline: 21.4288x; 1.0149x over previous
import functools

import numpy as np

import jax
import jax.numpy as jnp
from jax import lax
from jax.experimental import pallas as pl
from jax.experimental.pallas import tpu as pltpu


def _stage_geom(d, h, w):
    dp, hp, wp = d + 2, h + 2, w + 2
    hwp = hp * wp
    r = (d - 1) * hwp + (h - 1) * wp + (w - 1) + 1
    base = (np.arange(d)[:, None, None] * hwp
            + np.arange(h)[None, :, None] * wp
            + np.arange(w)[None, None, :])
    mask = np.zeros((1, r), np.float32)
    mask[0, base.reshape(-1)] = 1.0 / (d * h * w)
    return dp, hp, wp, r, base, mask


def _pool_mat(base, r, dq, hq, wq, dst_of):
    n_dst = int(np.max(dst_of)) + 1
    p = np.zeros((n_dst, r), np.float32)
    rows = dst_of.reshape(-1)
    for od in range(2):
        for oh in range(2):
            for ow in range(2):
                src = base[od:2 * dq:2, oh:2 * hq:2, ow:2 * wq:2].reshape(-1)
                p[rows, src] = 0.125
    return p


def _net_kernel(xf_ref, sc_ref, w0_ref, p0_ref, m0_ref, g0_ref, b0_ref,
                w1_ref, p1_ref, m1_ref, g1_ref, b1_ref,
                o_ref, xpad_ref, xf3_ref, acc0_ref, y0_ref, x1_ref, acc1_ref,
                *, d0, r0, d1, r1, s0, k1, hwp0, wp0, wp1, hw1, dq1, slab_k,
                nd0, m0rows, nd1, m1rows, hin, win, kc, scat, prows):
    xt = jnp.transpose(xf_ref[...], (1, 0))
    xpad_ref[...] = jnp.zeros(xpad_ref.shape, jnp.bfloat16)
    chunk = hin * win
    for dd in range(nd0):
        off = hwp0 * (dd + 1) + wp0 + 1
        fl, res = off - off % 16, off % 16
        sl = jnp.dot(sc_ref[scat[res]], xt[dd * chunk:(dd + 1) * chunk, :],
                     preferred_element_type=jnp.float32)
        xpad_ref[pl.ds(fl, prows), 0:kc] = sl.astype(jnp.bfloat16)
    for kw in range(3):
        xf3_ref[0:s0, kw * k1:(kw + 1) * k1] = xpad_ref[kw:kw + s0, :]
    for ds in range(nd0):
        base = ds * hwp0
        tot = None
        for t, dl in enumerate(d0):
            p = jnp.dot(xf3_ref[pl.ds(base + dl, m0rows), :], w0_ref[t],
                        preferred_element_type=jnp.float32)
            tot = p if tot is None else tot + p
        acc0_ref[pl.ds(base, m0rows), :] = tot
        if ds < nd0 - 1:
            acc0_ref[pl.ds(base + m0rows, hwp0 - m0rows), :] = (
                jnp.zeros((hwp0 - m0rows, tot.shape[1]), jnp.float32))
    acc = acc0_ref[...]
    mean = jnp.dot(m0_ref[...], acc, preferred_element_type=jnp.float32)
    ex2 = jnp.dot(m0_ref[...], acc * acc, preferred_element_type=jnp.float32)
    var = jnp.maximum(ex2 - mean * mean, 0.0)
    scale = g0_ref[...] * lax.rsqrt(var + 1e-5)
    shift = b0_ref[...] - mean * scale
    y0_ref[...] = jnp.maximum(acc * scale + shift, 0.0).astype(jnp.bfloat16)
    zero_slab = jnp.zeros((hw1, x1_ref.shape[1]), jnp.bfloat16)
    x1_ref[0:hw1, :] = zero_slab
    x1_ref[(dq1 + 1) * hw1:(dq1 + 2) * hw1, :] = zero_slab
    for od in range(dq1):
        sl = jnp.dot(p0_ref[...], y0_ref[pl.ds(2 * od * hwp0, slab_k), :],
                     preferred_element_type=jnp.float32)
        x1_ref[pl.ds((od + 1) * hw1, hw1), :] = sl.astype(jnp.bfloat16)

    for ds in range(nd1):
        base = ds * hw1
        tot = None
        for t, dl in enumerate(d1):
            p = jnp.dot(x1_ref[pl.ds(base + dl, m1rows), :], w1_ref[t],
                        preferred_element_type=jnp.float32)
            tot = p if tot is None else tot + p
        acc1_ref[pl.ds(base, m1rows), :] = tot
        if ds < nd1 - 1:
            acc1_ref[pl.ds(base + m1rows, hw1 - m1rows), :] = (
                jnp.zeros((hw1 - m1rows, tot.shape[1]), jnp.float32))
    acc1 = acc1_ref[...]
    mean1 = jnp.dot(m1_ref[...], acc1, preferred_element_type=jnp.float32)
    ex21 = jnp.dot(m1_ref[...], acc1 * acc1, preferred_element_type=jnp.float32)
    var1 = jnp.maximum(ex21 - mean1 * mean1, 0.0)
    scale1 = g1_ref[...] * lax.rsqrt(var1 + 1e-5)
    shift1 = b1_ref[...] - mean1 * scale1
    y1 = jnp.maximum(acc1 * scale1 + shift1, 0.0).astype(jnp.bfloat16)
    o_ref[...] = jnp.dot(p1_ref[...], y1, preferred_element_type=jnp.float32)


def _linear_kernel(a_ref, w_ref, b_ref, o_ref):
    o_ref[...] = (jnp.dot(a_ref[...], w_ref[...],
                          preferred_element_type=jnp.float32) + b_ref[...])


def kernel(x, l0_w, l0_b, l0_gamma, l0_beta,
           l1_w, l1_b, l1_gamma, l1_beta, cls_w, cls_b):
    n, c_in, d, h, w = (int(s) for s in x.shape)
    cw = int(l0_w.shape[0])
    grp = 16
    ng = n // grp
    lanes = grp * cw

    dp0, hp0, wp0, r0, base0, mask0 = _stage_geom(d, h, w)
    s0 = dp0 * hp0 * wp0
    d1, h1, w1 = d // 2, h // 2, w // 2
    dp1, hp1, wp1, r1, base1, mask1 = _stage_geom(d1, h1, w1)
    s1 = dp1 * hp1 * wp1
    d2, h2, w2 = d1 // 2, h1 // 2, w1 // 2
    s2 = d2 * h2 * w2

    deltas0 = tuple(kd * (hp0 * wp0) + kh * wp0
                    for kd in range(3) for kh in range(3))
    deltas1 = tuple(kd * (hp1 * wp1) + kh * wp1 + kw
                    for kd in range(3) for kh in range(3) for kw in range(3))

    hwp0 = hp0 * wp0
    hw1 = hp1 * wp1
    slab_k = hwp0 + (h - 1) * wp0 + (w - 1) + 1
    pool0 = np.zeros((hw1, slab_k), np.float32)
    for i in range(2):
        for j in range(2):
            for k in range(2):
                src = (i * hwp0 + (2 * np.arange(h1)[:, None] + j) * wp0
                       + 2 * np.arange(w1)[None, :] + k)
                dst = ((np.arange(h1)[:, None] + 1) * wp1
                       + np.arange(w1)[None, :] + 1)
                pool0[dst.reshape(-1), src.reshape(-1)] = 0.125
    dst1 = (np.arange(d2)[:, None, None] * (h2 * w2)
            + np.arange(h2)[None, :, None] * w2
            + np.arange(w2)[None, None, :])
    pool1 = _pool_mat(base1, r1, d2, h2, w2, dst1)

    k1 = 64
    kc = grp * c_in
    xf = x.astype(jnp.bfloat16).reshape(ng, kc, d * h * w)

    prows = (15 + (h - 1) * wp0 + w + 15) // 16 * 16
    res_list = sorted({(hwp0 * (dd + 1) + wp0 + 1) % 16 for dd in range(d)})
    scat = {r: i for i, r in enumerate(res_list)}
    scmat = np.zeros((len(res_list), prows, h * w), np.float32)
    for r, i in scat.items():
        hh = np.arange(h)[:, None]
        ww = np.arange(w)[None, :]
        scmat[i, (r + hh * wp0 + ww).reshape(-1),
              (hh * w + ww).reshape(-1)] = 1.0

    eye = jnp.eye(grp, dtype=jnp.float32)
    w0t = jnp.transpose(l0_w, (2, 3, 4, 1, 0)).reshape(9, 3, c_in, cw)
    w0bd = jnp.einsum('gh,tkio->tkgiho', eye, w0t)
    w0bd = w0bd.reshape(9, 3, grp * c_in, lanes)
    w0bd = jnp.pad(w0bd, ((0, 0), (0, 0), (0, k1 - grp * c_in), (0, 0)))
    w0bd = w0bd.reshape(9, 3 * k1, lanes).astype(jnp.bfloat16)
    w1t = jnp.transpose(l1_w, (2, 3, 4, 1, 0)).reshape(27, cw, cw)
    w1bd = jnp.einsum('gh,tio->tgiho', eye, w1t)
    w1bd = w1bd.reshape(27, grp * cw, lanes).astype(jnp.bfloat16)

    g0 = jnp.tile(l0_gamma, grp).reshape(1, lanes)
    bt0 = jnp.tile(l0_beta, grp).reshape(1, lanes)
    g1 = jnp.tile(l1_gamma, grp).reshape(1, lanes)
    bt1 = jnp.tile(l1_beta, grp).reshape(1, lanes)

    k_in = 3 * k1
    _body = functools.partial(_net_kernel, d0=deltas0, r0=r0,
                              d1=deltas1, r1=r1, s0=s0, k1=k1,
                              hwp0=hwp0, wp0=wp0, wp1=wp1, hw1=hw1, dq1=d1,
                              slab_k=slab_k,
                              nd0=d, m0rows=(h - 1) * wp0 + w,
                              nd1=d1, m1rows=(h1 - 1) * wp1 + w1,
                              hin=h, win=w, kc=kc, scat=scat, prows=prows)
    y = pl.pallas_call(
        _body,
        out_shape=jax.ShapeDtypeStruct((ng, s2, lanes), jnp.float32),
        grid=(ng,),
        in_specs=[
            pl.BlockSpec((None, kc, d * h * w), lambda i: (i, 0, 0)),
            pl.BlockSpec((len(res_list), prows, h * w), lambda i: (0, 0, 0)),
            pl.BlockSpec((9, k_in, lanes), lambda i: (0, 0, 0)),
            pl.BlockSpec((hw1, slab_k), lambda i: (0, 0)),
            pl.BlockSpec((1, r0), lambda i: (0, 0)),
            pl.BlockSpec((1, lanes), lambda i: (0, 0)),
            pl.BlockSpec((1, lanes), lambda i: (0, 0)),
            pl.BlockSpec((27, grp * cw, lanes), lambda i: (0, 0, 0)),
            pl.BlockSpec((s2, r1), lambda i: (0, 0)),
            pl.BlockSpec((1, r1), lambda i: (0, 0)),
            pl.BlockSpec((1, lanes), lambda i: (0, 0)),
            pl.BlockSpec((1, lanes), lambda i: (0, 0)),
        ],
        out_specs=pl.BlockSpec((None, s2, lanes), lambda i: (i, 0, 0)),
        scratch_shapes=[
            pltpu.VMEM((s0 + 2, k1), jnp.bfloat16),
            pltpu.VMEM((s0, k_in), jnp.bfloat16),
            pltpu.VMEM((r0, lanes), jnp.float32),
            pltpu.VMEM((r0, lanes), jnp.bfloat16),
            pltpu.VMEM((s1, lanes), jnp.bfloat16),
            pltpu.VMEM((r1, lanes), jnp.float32),
        ],
        compiler_params=pltpu.CompilerParams(
            dimension_semantics=("parallel",)),
    )(xf, jnp.asarray(scmat, jnp.bfloat16), w0bd, jnp.asarray(pool0, jnp.bfloat16),
      jnp.asarray(mask0), g0, bt0, w1bd,
      jnp.asarray(pool1, jnp.bfloat16), jnp.asarray(mask1), g1, bt1)

    nc = int(cls_w.shape[0])
    ncp = 128
    wc = cls_w.reshape(nc, cw, s2).transpose(2, 1, 0).reshape(s2 * cw, nc)
    wc = jnp.pad(wc, ((0, 0), (0, ncp - nc)))
    bc = jnp.pad(cls_b, (0, ncp - nc)).reshape(1, ncp)
    flat = y.reshape(ng, s2, grp, cw).transpose(0, 2, 1, 3).reshape(n, s2 * cw)
    logits = pl.pallas_call(
        _linear_kernel,
        out_shape=jax.ShapeDtypeStruct((n, ncp), jnp.float32),
        in_specs=[pl.BlockSpec(memory_space=pltpu.MemorySpace.VMEM)] * 3,
        out_specs=pl.BlockSpec(memory_space=pltpu.MemorySpace.VMEM),
    )(flat, wc, bc)
    return logits[:, :nc]


# speedup vs baseline: 21.9548x; 1.0245x over previous
import functools

import numpy as np

import jax
import jax.numpy as jnp
from jax import lax
from jax.experimental import pallas as pl
from jax.experimental.pallas import tpu as pltpu


def _stage_geom(d, h, w):
    dp, hp, wp = d + 2, h + 2, w + 2
    hwp = hp * wp
    r = (d - 1) * hwp + (h - 1) * wp + (w - 1) + 1
    base = (np.arange(d)[:, None, None] * hwp
            + np.arange(h)[None, :, None] * wp
            + np.arange(w)[None, None, :])
    mask = np.zeros((1, r), np.float32)
    mask[0, base.reshape(-1)] = 1.0 / (d * h * w)
    return dp, hp, wp, r, base, mask


def _pool_mat(base, r, dq, hq, wq, dst_of):
    n_dst = int(np.max(dst_of)) + 1
    p = np.zeros((n_dst, r), np.float32)
    rows = dst_of.reshape(-1)
    for od in range(2):
        for oh in range(2):
            for ow in range(2):
                src = base[od:2 * dq:2, oh:2 * hq:2, ow:2 * wq:2].reshape(-1)
                p[rows, src] = 0.125
    return p


def _net_kernel(xf_ref, sc_ref, w0_ref, p0_ref, m0_ref, g0_ref, b0_ref,
                w1_ref, p1_ref, m1_ref, g1_ref, b1_ref,
                o_ref, xpad_ref, xf3_ref, acc0_ref, y0_ref, x1_ref, acc1_ref,
                *, d0, r0, d1, r1, s0, k1, hwp0, wp0, wp1, hw1, dq1, slab_k,
                nd0, m0rows, nd1, m1rows, hin, win, kc, scat, prows):
    xt = jnp.transpose(xf_ref[...].astype(jnp.bfloat16), (1, 0))
    xpad_ref[...] = jnp.zeros(xpad_ref.shape, jnp.bfloat16)
    chunk = hin * win
    for dd in range(nd0):
        off = hwp0 * (dd + 1) + wp0 + 1
        fl, res = off - off % 16, off % 16
        sl = jnp.dot(sc_ref[scat[res]], xt[dd * chunk:(dd + 1) * chunk, :],
                     preferred_element_type=jnp.float32)
        xpad_ref[pl.ds(fl, prows), 0:kc] = sl.astype(jnp.bfloat16)
    for kw in range(3):
        xf3_ref[0:s0, kw * k1:(kw + 1) * k1] = xpad_ref[kw:kw + s0, :]
    for ds in range(nd0):
        base = ds * hwp0
        tot = None
        for t, dl in enumerate(d0):
            p = jnp.dot(xf3_ref[pl.ds(base + dl, m0rows), :], w0_ref[t],
                        preferred_element_type=jnp.float32)
            tot = p if tot is None else tot + p
        acc0_ref[pl.ds(base, m0rows), :] = tot
        if ds < nd0 - 1:
            acc0_ref[pl.ds(base + m0rows, hwp0 - m0rows), :] = (
                jnp.zeros((hwp0 - m0rows, tot.shape[1]), jnp.float32))
    acc = acc0_ref[...]
    mean = jnp.dot(m0_ref[...], acc, preferred_element_type=jnp.float32)
    ex2 = jnp.dot(m0_ref[...], acc * acc, preferred_element_type=jnp.float32)
    var = jnp.maximum(ex2 - mean * mean, 0.0)
    scale = g0_ref[...] * lax.rsqrt(var + 1e-5)
    shift = b0_ref[...] - mean * scale
    y0_ref[...] = jnp.maximum(acc * scale + shift, 0.0).astype(jnp.bfloat16)
    zero_slab = jnp.zeros((hw1, x1_ref.shape[1]), jnp.bfloat16)
    x1_ref[0:hw1, :] = zero_slab
    x1_ref[(dq1 + 1) * hw1:(dq1 + 2) * hw1, :] = zero_slab
    for od in range(dq1):
        sl = jnp.dot(p0_ref[...], y0_ref[pl.ds(2 * od * hwp0, slab_k), :],
                     preferred_element_type=jnp.float32)
        x1_ref[pl.ds((od + 1) * hw1, hw1), :] = sl.astype(jnp.bfloat16)

    for ds in range(nd1):
        base = ds * hw1
        tot = None
        for t, dl in enumerate(d1):
            p = jnp.dot(x1_ref[pl.ds(base + dl, m1rows), :], w1_ref[t],
                        preferred_element_type=jnp.float32)
            tot = p if tot is None else tot + p
        acc1_ref[pl.ds(base, m1rows), :] = tot
        if ds < nd1 - 1:
            acc1_ref[pl.ds(base + m1rows, hw1 - m1rows), :] = (
                jnp.zeros((hw1 - m1rows, tot.shape[1]), jnp.float32))
    acc1 = acc1_ref[...]
    mean1 = jnp.dot(m1_ref[...], acc1, preferred_element_type=jnp.float32)
    ex21 = jnp.dot(m1_ref[...], acc1 * acc1, preferred_element_type=jnp.float32)
    var1 = jnp.maximum(ex21 - mean1 * mean1, 0.0)
    scale1 = g1_ref[...] * lax.rsqrt(var1 + 1e-5)
    shift1 = b1_ref[...] - mean1 * scale1
    y1 = jnp.maximum(acc1 * scale1 + shift1, 0.0).astype(jnp.bfloat16)
    o_ref[...] = jnp.dot(p1_ref[...], y1, preferred_element_type=jnp.float32)


def _linear_kernel(a_ref, w_ref, b_ref, o_ref):
    o_ref[...] = (jnp.dot(a_ref[...], w_ref[...],
                          preferred_element_type=jnp.float32) + b_ref[...])


def kernel(x, l0_w, l0_b, l0_gamma, l0_beta,
           l1_w, l1_b, l1_gamma, l1_beta, cls_w, cls_b):
    n, c_in, d, h, w = (int(s) for s in x.shape)
    cw = int(l0_w.shape[0])
    grp = 16
    ng = n // grp
    lanes = grp * cw

    dp0, hp0, wp0, r0, base0, mask0 = _stage_geom(d, h, w)
    s0 = dp0 * hp0 * wp0
    d1, h1, w1 = d // 2, h // 2, w // 2
    dp1, hp1, wp1, r1, base1, mask1 = _stage_geom(d1, h1, w1)
    s1 = dp1 * hp1 * wp1
    d2, h2, w2 = d1 // 2, h1 // 2, w1 // 2
    s2 = d2 * h2 * w2

    deltas0 = tuple(kd * (hp0 * wp0) + kh * wp0
                    for kd in range(3) for kh in range(3))
    deltas1 = tuple(kd * (hp1 * wp1) + kh * wp1 + kw
                    for kd in range(3) for kh in range(3) for kw in range(3))

    hwp0 = hp0 * wp0
    hw1 = hp1 * wp1
    slab_k = hwp0 + (h - 1) * wp0 + (w - 1) + 1
    pool0 = np.zeros((hw1, slab_k), np.float32)
    for i in range(2):
        for j in range(2):
            for k in range(2):
                src = (i * hwp0 + (2 * np.arange(h1)[:, None] + j) * wp0
                       + 2 * np.arange(w1)[None, :] + k)
                dst = ((np.arange(h1)[:, None] + 1) * wp1
                       + np.arange(w1)[None, :] + 1)
                pool0[dst.reshape(-1), src.reshape(-1)] = 0.125
    dst1 = (np.arange(d2)[:, None, None] * (h2 * w2)
            + np.arange(h2)[None, :, None] * w2
            + np.arange(w2)[None, None, :])
    pool1 = _pool_mat(base1, r1, d2, h2, w2, dst1)

    k1 = 64
    kc = grp * c_in
    xf = x.reshape(ng, kc, d * h * w)

    prows = (15 + (h - 1) * wp0 + w + 15) // 16 * 16
    res_list = sorted({(hwp0 * (dd + 1) + wp0 + 1) % 16 for dd in range(d)})
    scat = {r: i for i, r in enumerate(res_list)}
    scmat = np.zeros((len(res_list), prows, h * w), np.float32)
    for r, i in scat.items():
        hh = np.arange(h)[:, None]
        ww = np.arange(w)[None, :]
        scmat[i, (r + hh * wp0 + ww).reshape(-1),
              (hh * w + ww).reshape(-1)] = 1.0

    eye = jnp.eye(grp, dtype=jnp.float32)
    w0t = jnp.transpose(l0_w, (2, 3, 4, 1, 0)).reshape(9, 3, c_in, cw)
    w0bd = jnp.einsum('gh,tkio->tkgiho', eye, w0t)
    w0bd = w0bd.reshape(9, 3, grp * c_in, lanes)
    w0bd = jnp.pad(w0bd, ((0, 0), (0, 0), (0, k1 - grp * c_in), (0, 0)))
    w0bd = w0bd.reshape(9, 3 * k1, lanes).astype(jnp.bfloat16)
    w1t = jnp.transpose(l1_w, (2, 3, 4, 1, 0)).reshape(27, cw, cw)
    w1bd = jnp.einsum('gh,tio->tgiho', eye, w1t)
    w1bd = w1bd.reshape(27, grp * cw, lanes).astype(jnp.bfloat16)

    g0 = jnp.tile(l0_gamma, grp).reshape(1, lanes)
    bt0 = jnp.tile(l0_beta, grp).reshape(1, lanes)
    g1 = jnp.tile(l1_gamma, grp).reshape(1, lanes)
    bt1 = jnp.tile(l1_beta, grp).reshape(1, lanes)

    k_in = 3 * k1
    _body = functools.partial(_net_kernel, d0=deltas0, r0=r0,
                              d1=deltas1, r1=r1, s0=s0, k1=k1,
                              hwp0=hwp0, wp0=wp0, wp1=wp1, hw1=hw1, dq1=d1,
                              slab_k=slab_k,
                              nd0=d, m0rows=(h - 1) * wp0 + w,
                              nd1=d1, m1rows=(h1 - 1) * wp1 + w1,
                              hin=h, win=w, kc=kc, scat=scat, prows=prows)
    y = pl.pallas_call(
        _body,
        out_shape=jax.ShapeDtypeStruct((ng, s2, lanes), jnp.float32),
        grid=(ng,),
        in_specs=[
            pl.BlockSpec((None, kc, d * h * w), lambda i: (i, 0, 0)),
            pl.BlockSpec((len(res_list), prows, h * w), lambda i: (0, 0, 0)),
            pl.BlockSpec((9, k_in, lanes), lambda i: (0, 0, 0)),
            pl.BlockSpec((hw1, slab_k), lambda i: (0, 0)),
            pl.BlockSpec((1, r0), lambda i: (0, 0)),
            pl.BlockSpec((1, lanes), lambda i: (0, 0)),
            pl.BlockSpec((1, lanes), lambda i: (0, 0)),
            pl.BlockSpec((27, grp * cw, lanes), lambda i: (0, 0, 0)),
            pl.BlockSpec((s2, r1), lambda i: (0, 0)),
            pl.BlockSpec((1, r1), lambda i: (0, 0)),
            pl.BlockSpec((1, lanes), lambda i: (0, 0)),
            pl.BlockSpec((1, lanes), lambda i: (0, 0)),
        ],
        out_specs=pl.BlockSpec((None, s2, lanes), lambda i: (i, 0, 0)),
        scratch_shapes=[
            pltpu.VMEM((s0 + 2, k1), jnp.bfloat16),
            pltpu.VMEM((s0, k_in), jnp.bfloat16),
            pltpu.VMEM((r0, lanes), jnp.float32),
            pltpu.VMEM((r0, lanes), jnp.bfloat16),
            pltpu.VMEM((s1, lanes), jnp.bfloat16),
            pltpu.VMEM((r1, lanes), jnp.float32),
        ],
        compiler_params=pltpu.CompilerParams(
            dimension_semantics=("parallel",)),
    )(xf, jnp.asarray(scmat, jnp.bfloat16), w0bd, jnp.asarray(pool0, jnp.bfloat16),
      jnp.asarray(mask0), g0, bt0, w1bd,
      jnp.asarray(pool1, jnp.bfloat16), jnp.asarray(mask1), g1, bt1)

    nc = int(cls_w.shape[0])
    ncp = 128
    wc = cls_w.reshape(nc, cw, s2).transpose(2, 1, 0).reshape(s2 * cw, nc)
    wc = jnp.pad(wc, ((0, 0), (0, ncp - nc)))
    bc = jnp.pad(cls_b, (0, ncp - nc)).reshape(1, ncp)
    flat = y.reshape(ng, s2, grp, cw).transpose(0, 2, 1, 3).reshape(n, s2 * cw)
    logits = pl.pallas_call(
        _linear_kernel,
        out_shape=jax.ShapeDtypeStruct((n, ncp), jnp.float32),
        in_specs=[pl.BlockSpec(memory_space=pltpu.MemorySpace.VMEM)] * 3,
        out_specs=pl.BlockSpec(memory_space=pltpu.MemorySpace.VMEM),
    )(flat, wc, bc)
    return logits[:, :nc]
